# baseline (speedup 1.0000x reference)
import jax
import jax.numpy as jnp
from jax import lax
from jax.experimental import pallas as pl
from jax.experimental.pallas import tpu as pltpu
from jax.experimental.pallas import tpu_sc as plsc

_B, _S, _D = 4, 8192, 768
_NC, _NS, _L = 2, 16, 16
_TPB = (_NC * _NS) // _B
_CHUNK = _S // _TPB
_G = 64
_NV = _D // _L


def _sc_body(feat_hbm, mask_hbm, out_hbm, mask_v, idx_v, rows_v, acc_v,
             cnt_v, sums_v, cnts_v, res_v, shared_sum, shared_cnt):
    c = lax.axis_index("c")
    s = lax.axis_index("s")
    b = c * (_NS // _TPB) + s // _TPB
    seg = s % _TPB
    moff = pl.multiple_of(b * _S + seg * _CHUNK, _CHUNK)

    pltpu.sync_copy(mask_hbm.at[pl.ds(moff, _CHUNK)], mask_v)

    zf = jnp.zeros((_L,), jnp.float32)
    zi = jnp.zeros((_L,), jnp.int32)
    for j in range(_NV):
        acc_v[pl.ds(j * _L, _L)] = zf
    for j in range(idx_v.shape[0] // _L):
        idx_v[pl.ds(j * _L, _L)] = zi

    iota = lax.iota(jnp.int32, _L)
    lane_last = jnp.full((_L,), _L - 1, jnp.int32)
    shift_src = [jnp.maximum(iota - sh, 0) for sh in (1, 2, 4, 8)]
    shift_keep = [iota >= sh for sh in (1, 2, 4, 8)]
    zero16 = jnp.zeros((_L,), jnp.int32)
    cnt_vec = zero16
    for j in range(_CHUNK // _L):
        mv = mask_v[pl.ds(j * _L, _L)]
        msk = mv > 0
        vals = moff + j * _L + iota
        ranks = mv
        for src, keep in zip(shift_src, shift_keep):
            shifted = ranks.at[src].get(mode="promise_in_bounds")
            ranks = ranks + jnp.where(keep, shifted, 0)
        pos = jnp.where(msk, cnt_vec + ranks - 1, _CHUNK + iota)
        plsc.store_scatter(idx_v, [pos], vals)
        cnt_vec = cnt_vec + ranks.at[lane_last].get(mode="promise_in_bounds")
    cnt = cnt_vec[0]

    nb = (cnt + _G - 1) // _G

    def blk_body(blk, _):
        off = pl.multiple_of(blk * _G, _G)
        pltpu.sync_copy(feat_hbm.at[idx_v.at[pl.ds(off, _G)]], rows_v)
        nrows = jnp.minimum(_G, cnt - off)

        def row_body(r, _):
            for j in range(_NV):
                sl = pl.ds(j * _L, _L)
                plsc.addupdate(acc_v.at[sl], rows_v[r, sl])
            return 0

        lax.fori_loop(0, nrows, row_body, 0, unroll=False)
        return 0

    lax.fori_loop(0, nb, blk_body, 0, unroll=False)

    cnt_v[...] = jnp.full((_L,), 1.0, jnp.float32) * cnt.astype(jnp.float32)
    pltpu.sync_copy(acc_v, shared_sum.at[pl.ds(pl.multiple_of(s * _D, 8), _D)])
    pltpu.sync_copy(cnt_v, shared_cnt.at[pl.ds(pl.multiple_of(s * _L, 8), _L)])
    plsc.subcore_barrier()

    @pl.when(seg == 0)
    def _combine():
        pltpu.sync_copy(
            shared_sum.at[pl.ds(pl.multiple_of(s * _D, 8), _TPB * _D)], sums_v)
        pltpu.sync_copy(
            shared_cnt.at[pl.ds(pl.multiple_of(s * _L, 8), _TPB * _L)], cnts_v)
        tot = cnts_v[pl.ds(0, _L)]
        for r in range(1, _TPB):
            tot = tot + cnts_v[pl.ds(r * _L, _L)]
        inv = 1.0 / jnp.maximum(tot, 1.0)
        for j in range(_NV):
            v = sums_v[pl.ds(j * _L, _L)]
            for r in range(1, _TPB):
                v = v + sums_v[pl.ds(r * _D + j * _L, _L)]
            res_v[pl.ds(j * _L, _L)] = v * inv
        pltpu.sync_copy(
            res_v, out_hbm.at[pl.ds(pl.multiple_of(b * _D, 8), _D)])


def kernel(features, mask):
    B, S, D = features.shape
    feat2d = features.reshape(B * S, D)
    mask_i = mask.astype(jnp.int32).reshape(B * S)
    mesh = plsc.VectorSubcoreMesh(
        core_axis_name="c", subcore_axis_name="s",
        num_cores=_NC, num_subcores=_NS,
    )
    f = pl.kernel(
        _sc_body,
        out_type=jax.ShapeDtypeStruct((B * D,), jnp.float32),
        mesh=mesh,
        compiler_params=pltpu.CompilerParams(needs_layout_passes=False),
        scratch_types=[
            pltpu.VMEM((_CHUNK,), jnp.int32),
            pltpu.VMEM((_CHUNK + _L,), jnp.int32),
            pltpu.VMEM((_G, _D), jnp.float32),
            pltpu.VMEM((_D,), jnp.float32),
            pltpu.VMEM((_L,), jnp.float32),
            pltpu.VMEM((_TPB * _D,), jnp.float32),
            pltpu.VMEM((_TPB * _L,), jnp.float32),
            pltpu.VMEM((_D,), jnp.float32),
            pltpu.VMEM_SHARED((_NS * _D,), jnp.float32),
            pltpu.VMEM_SHARED((_NS * _L,), jnp.float32),
        ],
    )
    return f(feat2d, mask_i).reshape(B, D)


# speedup vs baseline: 2.5747x; 2.5747x over previous
import jax
import jax.numpy as jnp
from jax import lax
from jax.experimental import pallas as pl
from jax.experimental.pallas import tpu as pltpu
from jax.experimental.pallas import tpu_sc as plsc

_B, _S, _D = 4, 8192, 768
_NC, _NS, _L = 2, 16, 16
_TPB = (_NC * _NS) // _B
_X = 2560
_CH = 512
_CHUNK = _X // _TPB
_G = 16
_NBUF = 4
_NV = _D // _L


def _tree_sum(terms):
    while len(terms) > 1:
        nxt = [a + b for a, b in zip(terms[::2], terms[1::2])]
        if len(terms) % 2:
            nxt[-1] = nxt[-1] + terms[-1]
        terms = nxt
    return terms[0]


def _sc_body(feat_hbm, mask_hbm, sum_hbm, cnt_hbm, mask_v, idx_v, w_v,
             rows_v0, rows_v1, rows_v2, rows_v3, acc_v, cnt_v, sums_v,
             cnts_v, res_v, cres_v, shared_sum, shared_cnt,
             sem0, sem1, sem2, sem3):
    c = lax.axis_index("c")
    s = lax.axis_index("s")
    b = c * (_NS // _TPB) + s // _TPB
    seg = s % _TPB
    moff = pl.multiple_of(b * _S + seg * _CHUNK, _CHUNK)

    pltpu.sync_copy(mask_hbm.at[pl.ds(moff, _CHUNK)], mask_v)

    zf = jnp.zeros((_L,), jnp.float32)
    zi = jnp.zeros((_L,), jnp.int32)
    for j in range(_NV):
        acc_v[pl.ds(j * _L, _L)] = zf
    for j in range(idx_v.shape[0] // _L):
        idx_v[pl.ds(j * _L, _L)] = zi

    iota = lax.iota(jnp.int32, _L)
    lane_last = jnp.full((_L,), _L - 1, jnp.int32)
    shift_src = [jnp.maximum(iota - sh, 0) for sh in (1, 2, 4, 8)]
    shift_keep = [iota >= sh for sh in (1, 2, 4, 8)]
    cnt_vec = jnp.zeros((_L,), jnp.int32)
    for j in range(_CHUNK // _L):
        mv = mask_v[pl.ds(j * _L, _L)]
        msk = mv > 0
        vals = moff + j * _L + iota
        ranks = mv
        for src, keep in zip(shift_src, shift_keep):
            shifted = ranks.at[src].get(mode="promise_in_bounds")
            ranks = ranks + jnp.where(keep, shifted, 0)
        pos = jnp.where(msk, cnt_vec + ranks - 1, _CHUNK + iota)
        plsc.store_scatter(idx_v, [pos], vals)
        cnt_vec = cnt_vec + ranks.at[lane_last].get(mode="promise_in_bounds")
    cnt = cnt_vec[0]

    for j in range(_CHUNK // _L):
        g_vec = j * _L + iota
        w_v[pl.ds(j * _L, _L)] = jnp.where(g_vec < cnt_vec, 1.0, 0.0)

    nb = (cnt + _G - 1) // _G
    lane_consts = [jnp.full((_L,), r, jnp.int32) for r in range(_L)]
    bufs = [rows_v0, rows_v1, rows_v2, rows_v3]
    sems = [sem0, sem1, sem2, sem3]

    def gather(blk, t):
        off = pl.multiple_of(blk * _G, _G)
        return pltpu.make_async_copy(
            feat_hbm.at[idx_v.at[pl.ds(off, _G)]], bufs[t], sems[t])

    for t in range(_NBUF):
        @pl.when(t < nb)
        def _prime(t=t):
            gather(jnp.int32(t), t).start()

    def outer_body(o, _):
        for t in range(_NBUF):
            blk = o * _NBUF + t

            @pl.when(blk < nb)
            def _step(blk=blk, t=t):
                gather(blk, t).wait()
                off = pl.multiple_of(blk * _G, _G)
                buf = bufs[t]
                wv = [w_v[pl.ds(off + h * _L, _L)] for h in range(_G // _L)]
                for j in range(_NV):
                    sl = pl.ds(j * _L, _L)
                    terms = [
                        buf[r, sl]
                        * wv[r // _L].at[lane_consts[r % _L]].get(
                            mode="promise_in_bounds")
                        for r in range(_G)
                    ]
                    plsc.addupdate(acc_v.at[sl], _tree_sum(terms))

                @pl.when(blk + _NBUF < nb)
                def _fire_ahead():
                    gather(blk + _NBUF, t).start()
        return 0

    lax.fori_loop(0, (nb + _NBUF - 1) // _NBUF, outer_body, 0, unroll=False)

    cnt_v[...] = jnp.full((_L,), 1.0, jnp.float32) * cnt.astype(jnp.float32)
    pltpu.sync_copy(acc_v, shared_sum.at[pl.ds(pl.multiple_of(s * _D, 8), _D)])
    pltpu.sync_copy(cnt_v, shared_cnt.at[pl.ds(pl.multiple_of(s * _L, 8), _L)])
    plsc.subcore_barrier()

    @pl.when(seg == 0)
    def _combine():
        pltpu.sync_copy(
            shared_sum.at[pl.ds(pl.multiple_of(s * _D, 8), _TPB * _D)], sums_v)
        pltpu.sync_copy(
            shared_cnt.at[pl.ds(pl.multiple_of(s * _L, 8), _TPB * _L)], cnts_v)
        tot = _tree_sum([cnts_v[pl.ds(r * _L, _L)] for r in range(_TPB)])
        for j in range(_NV):
            v = _tree_sum(
                [sums_v[pl.ds(r * _D + j * _L, _L)] for r in range(_TPB)])
            res_v[pl.ds(j * _L, _L)] = v
            cres_v[pl.ds(j * _L, _L)] = tot
        pltpu.sync_copy(
            res_v, sum_hbm.at[pl.ds(pl.multiple_of(b * _D, 8), _D)])
        pltpu.sync_copy(
            cres_v, cnt_hbm.at[pl.ds(pl.multiple_of(b * _D, 8), _D)])


def _sc_partial(feat2d, mask_i):
    mesh = plsc.VectorSubcoreMesh(
        core_axis_name="c", subcore_axis_name="s",
        num_cores=_NC, num_subcores=_NS,
    )
    f = pl.kernel(
        _sc_body,
        out_type=[
            jax.ShapeDtypeStruct((_B * _D,), jnp.float32),
            jax.ShapeDtypeStruct((_B * _D,), jnp.float32),
        ],
        mesh=mesh,
        compiler_params=pltpu.CompilerParams(needs_layout_passes=False),
        scratch_types=[
            pltpu.VMEM((_CHUNK,), jnp.int32),
            pltpu.VMEM((_CHUNK + _L,), jnp.int32),
            pltpu.VMEM((_CHUNK,), jnp.float32),
            pltpu.VMEM((_G, _D), jnp.float32),
            pltpu.VMEM((_G, _D), jnp.float32),
            pltpu.VMEM((_G, _D), jnp.float32),
            pltpu.VMEM((_G, _D), jnp.float32),
            pltpu.VMEM((_D,), jnp.float32),
            pltpu.VMEM((_L,), jnp.float32),
            pltpu.VMEM((_TPB * _D,), jnp.float32),
            pltpu.VMEM((_TPB * _L,), jnp.float32),
            pltpu.VMEM((_D,), jnp.float32),
            pltpu.VMEM((_D,), jnp.float32),
            pltpu.VMEM_SHARED((_NS * _D,), jnp.float32),
            pltpu.VMEM_SHARED((_NS * _L,), jnp.float32),
            pltpu.SemaphoreType.DMA,
            pltpu.SemaphoreType.DMA,
            pltpu.SemaphoreType.DMA,
            pltpu.SemaphoreType.DMA,
        ],
    )
    return f(feat2d, mask_i)


def _tc_body(m_ref, f_ref, osum_ref, ocnt_ref, acc_ref, cnt_ref):
    j = pl.program_id(1)
    nj = pl.num_programs(1)

    @pl.when(j == 0)
    def _init():
        acc_ref[...] = jnp.zeros_like(acc_ref)
        cnt_ref[0] = 0.0

    m = m_ref[...]
    f = f_ref[...]
    acc_ref[...] += jnp.sum(f * m[0, 0, 0][:, None], axis=1)
    cnt_ref[0] += jnp.sum(m)

    @pl.when(j == nj - 1)
    def _final():
        osum_ref[...] = acc_ref[...][None]
        ocnt_ref[...] = jnp.full(ocnt_ref.shape, cnt_ref[0], jnp.float32)


def _tc_partial(maskf4d, features):
    xc = _X // _CH
    nch = _S // _CH - xc
    return pl.pallas_call(
        _tc_body,
        grid=(_B, nch),
        in_specs=[
            pl.BlockSpec((1, 1, 1, _CH), lambda i, j: (i, j + xc, 0, 0)),
            pl.BlockSpec((1, _CH, _D), lambda i, j: (i, j + xc, 0)),
        ],
        out_specs=[
            pl.BlockSpec((1, 1, _D), lambda i, j: (i, 0, 0)),
            pl.BlockSpec((1, 1, _D), lambda i, j: (i, 0, 0)),
        ],
        out_shape=[
            jax.ShapeDtypeStruct((_B, 1, _D), jnp.float32),
            jax.ShapeDtypeStruct((_B, 1, _D), jnp.float32),
        ],
        scratch_shapes=[
            pltpu.VMEM((1, _D), jnp.float32),
            pltpu.SMEM((1,), jnp.float32),
        ],
        compiler_params=pltpu.CompilerParams(
            dimension_semantics=("parallel", "arbitrary"),
        ),
    )(maskf4d, features)


def _merge_body(a_ref, b_ref, ca_ref, cb_ref, o_ref):
    tot = jnp.maximum(ca_ref[...] + cb_ref[...], 1.0)
    o_ref[...] = (a_ref[...] + b_ref[...]) / tot


def _merge(sc_sum, tc_sum, sc_cnt, tc_cnt):
    return pl.pallas_call(
        _merge_body,
        out_shape=jax.ShapeDtypeStruct((_B, 1, _D), jnp.float32),
    )(sc_sum, tc_sum, sc_cnt, tc_cnt)


def kernel(features, mask):
    B, S, D = features.shape
    feat2d = features.reshape(B * S, D)
    mask_i = mask.astype(jnp.int32).reshape(B * S)
    maskf4d = mask.astype(jnp.float32).reshape(B, S // _CH, 1, _CH)
    sc_sum, sc_cnt = _sc_partial(feat2d, mask_i)
    tc_sum, tc_cnt = _tc_partial(maskf4d, features)
    out = _merge(sc_sum.reshape(B, 1, D), tc_sum,
                 sc_cnt.reshape(B, 1, D), tc_cnt)
    return out.reshape(B, D)
